# baseline (speedup 1.0000x reference)
import jax
import jax.numpy as jnp
from jax.experimental import pallas as pl

N_NODES = 50000
NUM_G = 64


def _mm_bias_kernel(x_ref, w_ref, b_ref, o_ref):
    o_ref[...] = jnp.dot(x_ref[...], w_ref[...],
                         preferred_element_type=jnp.float32) + b_ref[...]


def _mm_bias(x, w, b):
    n, k = x.shape
    f = w.shape[1]
    blk = 2000
    return pl.pallas_call(
        _mm_bias_kernel,
        grid=(n // blk,),
        in_specs=[
            pl.BlockSpec((blk, k), lambda i: (i, 0)),
            pl.BlockSpec((k, f), lambda i: (0, 0)),
            pl.BlockSpec((f,), lambda i: (0,)),
        ],
        out_specs=pl.BlockSpec((blk, f), lambda i: (i, 0)),
        out_shape=jax.ShapeDtypeStruct((n, f), jnp.float32),
    )(x, w, b)


def _mpnn(x, src, dst, W, b, n):
    agg = jax.ops.segment_sum(x[src], dst, num_segments=n)
    return _mm_bias(agg, W, b)


def _gcn(x, src, dst, W, b, n):
    sl = jnp.arange(n, dtype=src.dtype)
    s = jnp.concatenate([src, sl])
    d = jnp.concatenate([dst, sl])
    deg = jax.ops.segment_sum(jnp.ones(d.shape[0], dtype=x.dtype), d, num_segments=n)
    dis = jnp.where(deg > 0, 1.0 / jnp.sqrt(deg), 0.0)
    norm = dis[s] * dis[d]
    xw = _mm_bias(x, W, jnp.zeros((W.shape[1],), jnp.float32))
    out = jax.ops.segment_sum(norm[:, None] * xw[s], d, num_segments=n)
    return out + b


def _att_pool(z, batch, W1, b1, W2, b2, num_graphs):
    gate = _mm_bias(jnp.maximum(_mm_bias(z, W1, b1), 0.0), W2, b2)
    gmax = jax.ops.segment_max(gate, batch, num_segments=num_graphs)
    gmax = jnp.where(jnp.isfinite(gmax), gmax, 0.0)
    e = jnp.exp(gate - gmax[batch])
    denom = jax.ops.segment_sum(e, batch, num_segments=num_graphs)
    att = e / (denom[batch] + 1e-16)
    return jax.ops.segment_sum(att * z, batch, num_segments=num_graphs)


def kernel(x1, x2, pe, edge_index, batch, W_ec, b_ec, W_mf, b_mf, W_gf, b_gf,
           W_mc, b_mc, W_gc, b_gc, W_a1, b_a1, W_a2, b_a2, W_gd, b_gd,
           W_d1, b_d1, W_d2, b_d2, W_d3, b_d3):
    src, dst = edge_index[0], edge_index[1]
    n = x1.shape[0]
    pe3 = jnp.concatenate([pe, pe, pe], axis=-1)
    h = _mm_bias(x1, W_ec, b_ec)
    h_res = jax.nn.relu(h)
    h = h_res + pe3
    h = jax.nn.relu(_mpnn(h, src, dst, W_mf, b_mf, n))
    h = jax.nn.relu(_gcn(h, src, dst, W_gf, b_gf, n))
    c = jax.nn.relu(_mpnn(x2, src, dst, W_mc, b_mc, n))
    c = jax.nn.relu(_gcn(c, src, dst, W_gc, b_gc, n))
    z = h + c + h_res
    z = _att_pool(z, batch, W_a1, b_a1, W_a2, b_a2, NUM_G)
    z1 = z[batch]
    z1 = _gcn(z1, src, dst, W_gd, b_gd, n)
    z1_res = jax.nn.relu(z1)
    z1 = z1_res + pe3
    z1 = jax.nn.relu(_mpnn(z1, src, dst, W_d1, b_d1, n))
    z1 = jax.nn.relu(_mpnn(z1, src, dst, W_d2, b_d2, n)) + z1_res
    z1 = jax.nn.relu(_mpnn(z1, src, dst, W_d3, b_d3, n))
    return z1


# speedup vs baseline: 4.9158x; 4.9158x over previous
import functools

import jax
import jax.numpy as jnp
from jax import lax
from jax.experimental import pallas as pl
from jax.experimental.pallas import tpu as pltpu, tpu_sc as plsc

N = 50000
E = 1600000
G = 64
F = 48
H = 24
NPAD = 50048
ZR = NPAD // 16
CH = 80
BLK = 2000
GRID = N // BLK
NEG = -3.0e38


def _f32(*shape):
    return jax.ShapeDtypeStruct(shape, jnp.float32)


@functools.cache
def _mesh():
    return plsc.VectorSubcoreMesh(core_axis_name="c", subcore_axis_name="s")


def _agg48_body(xlo, xhi, src, dst, zeros, out_lo, out_hi,
                idx_s, idx_d, rows, acc, sem):
    c = lax.axis_index("c")
    t = lax.axis_index("s")
    base_r = t * ZR
    pltpu.sync_copy(zeros, acc.at[pl.ds(base_r, ZR)])
    plsc.subcore_barrier()
    ept = E // 16
    ebase = t * ept

    def chunk(i, carry):
        off = ebase + i * CH
        pltpu.sync_copy(src.at[pl.ds(off, CH)], idx_s)
        pltpu.sync_copy(dst.at[pl.ds(off, CH)], idx_d)

        @pl.when(c == 0)
        def _():
            pltpu.async_copy(xlo.at[idx_s], rows, sem).wait()

        @pl.when(c == 1)
        def _():
            pltpu.async_copy(xhi.at[idx_s], rows, sem).wait()

        pltpu.sync_copy(rows, acc.at[idx_d], add=True)
        return carry

    lax.fori_loop(0, ept // CH, chunk, 0)
    plsc.subcore_barrier()

    @pl.when(c == 0)
    def _():
        pltpu.sync_copy(acc.at[pl.ds(base_r, ZR)], out_lo.at[pl.ds(base_r, ZR)])

    @pl.when(c == 1)
    def _():
        pltpu.sync_copy(acc.at[pl.ds(base_r, ZR)], out_hi.at[pl.ds(base_r, ZR)])


@functools.cache
def _agg48_k():
  return pl.kernel(
    _agg48_body,
    out_type=(_f32(NPAD, H), _f32(NPAD, H)),
    mesh=_mesh(),
    scratch_types=[
        pltpu.VMEM((CH,), jnp.int32),
        pltpu.VMEM((CH,), jnp.int32),
        pltpu.VMEM((CH, H), jnp.float32),
        pltpu.VMEM_SHARED((NPAD, H), jnp.float32),
        pltpu.SemaphoreType.DMA,
    ],
    compiler_params=pltpu.CompilerParams(use_tc_tiling_on_sc=False),
)


def _agg16_body(x2, src, dst, zeros, out0, out1,
                idx_s, idx_d, rows, acc, sem):
    c = lax.axis_index("c")
    t = lax.axis_index("s")
    base_r = t * ZR
    pltpu.sync_copy(zeros, acc.at[pl.ds(base_r, ZR)])
    plsc.subcore_barrier()
    ept = E // 32
    ebase = c * (E // 2) + t * ept

    def chunk(i, carry):
        off = ebase + i * CH
        pltpu.sync_copy(src.at[pl.ds(off, CH)], idx_s)
        pltpu.sync_copy(dst.at[pl.ds(off, CH)], idx_d)
        pltpu.async_copy(x2.at[idx_s], rows, sem).wait()
        pltpu.sync_copy(rows, acc.at[idx_d], add=True)
        return carry

    lax.fori_loop(0, ept // CH, chunk, 0)
    plsc.subcore_barrier()

    @pl.when(c == 0)
    def _():
        pltpu.sync_copy(acc.at[pl.ds(base_r, ZR)], out0.at[pl.ds(base_r, ZR)])

    @pl.when(c == 1)
    def _():
        pltpu.sync_copy(acc.at[pl.ds(base_r, ZR)], out1.at[pl.ds(base_r, ZR)])


@functools.cache
def _agg16_k():
  return pl.kernel(
    _agg16_body,
    out_type=(_f32(NPAD, 16), _f32(NPAD, 16)),
    mesh=_mesh(),
    scratch_types=[
        pltpu.VMEM((CH,), jnp.int32),
        pltpu.VMEM((CH,), jnp.int32),
        pltpu.VMEM((CH, 16), jnp.float32),
        pltpu.VMEM_SHARED((NPAD, 16), jnp.float32),
        pltpu.SemaphoreType.DMA,
    ],
    compiler_params=pltpu.CompilerParams(use_tc_tiling_on_sc=False),
)


def _deg_body(dst, zeros, out0, out1, idx_d, ones, acc):
    c = lax.axis_index("c")
    t = lax.axis_index("s")
    base_r = t * ZR
    for j in range(CH // 16):
        ones[pl.ds(j * 16, 16)] = jnp.full((16,), 1.0, jnp.float32)
    pltpu.sync_copy(zeros, acc.at[pl.ds(base_r, ZR)])
    plsc.subcore_barrier()
    ept = E // 32
    ebase = c * (E // 2) + t * ept

    def chunk(i, carry):
        off = ebase + i * CH
        pltpu.sync_copy(dst.at[pl.ds(off, CH)], idx_d)
        pltpu.sync_copy(ones, acc.at[idx_d], add=True)
        return carry

    lax.fori_loop(0, ept // CH, chunk, 0)
    plsc.subcore_barrier()

    @pl.when(c == 0)
    def _():
        pltpu.sync_copy(acc.at[pl.ds(base_r, ZR)], out0.at[pl.ds(base_r, ZR)])

    @pl.when(c == 1)
    def _():
        pltpu.sync_copy(acc.at[pl.ds(base_r, ZR)], out1.at[pl.ds(base_r, ZR)])


@functools.cache
def _deg_k():
  return pl.kernel(
    _deg_body,
    out_type=(_f32(NPAD), _f32(NPAD)),
    mesh=_mesh(),
    scratch_types=[
        pltpu.VMEM((CH,), jnp.int32),
        pltpu.VMEM((CH,), jnp.float32),
        pltpu.VMEM_SHARED((NPAD,), jnp.float32),
    ],
    compiler_params=pltpu.CompilerParams(use_tc_tiling_on_sc=False),
)


def _row_spec(w):
    return pl.BlockSpec((BLK, w), lambda i: (i, 0))


def _full_spec(*shape):
    nd = len(shape)
    return pl.BlockSpec(shape, lambda i, _nd=nd: (0,) * nd)


def _tc_call(body, in_specs, out_specs, out_shape):
    return pl.pallas_call(body, grid=(GRID,), in_specs=in_specs,
                          out_specs=out_specs, out_shape=out_shape)


def _dis(p0_ref, p1_ref):
    deg = p0_ref[...] + p1_ref[...] + 1.0
    return lax.rsqrt(deg), 1.0 / deg


def _enc_body(x1, pe, w, b, h_res, lo, hi):
    h = jnp.dot(x1[...], w[...], preferred_element_type=jnp.float32) + b[...]
    hr = jnp.maximum(h, 0.0)
    h_res[...] = hr
    p = pe[...]
    hin = hr + jnp.concatenate([p, p, p], axis=1)
    lo[...] = hin[:, :H]
    hi[...] = hin[:, H:]


def _encoder(x1, pe, w, b):
    return _tc_call(
        _enc_body,
        [_row_spec(128), _row_spec(16), _full_spec(128, F), _full_spec(F)],
        (_row_spec(F), _row_spec(H), _row_spec(H)),
        (_f32(N, F), _f32(NPAD, H), _f32(NPAD, H)),
    )(x1, pe, w, b)


def _mlp_gcnprep_body(alo, ahi, p0, p1, wm, bm, wg, xw_o, ylo, yhi):
    t = (jnp.dot(alo[...], wm[:H, :], preferred_element_type=jnp.float32)
         + jnp.dot(ahi[...], wm[H:, :], preferred_element_type=jnp.float32)
         + bm[...])
    t = jnp.maximum(t, 0.0)
    xw = jnp.dot(t, wg[...], preferred_element_type=jnp.float32)
    dis, _ = _dis(p0, p1)
    y = dis * xw
    xw_o[...] = xw
    ylo[...] = y[:, :H]
    yhi[...] = y[:, H:]


def _mlp_gcnprep(alo, ahi, p0, p1, wm, bm, wg, kin):
    return _tc_call(
        _mlp_gcnprep_body,
        [_row_spec(H), _row_spec(H), _row_spec(1), _row_spec(1),
         _full_spec(kin, F), _full_spec(F), _full_spec(F, F)],
        (_row_spec(F), _row_spec(H), _row_spec(H)),
        (_f32(N, F), _f32(NPAD, H), _f32(NPAD, H)),
    )(alo, ahi, p0, p1, wm, bm, wg)


def _mlp16_body(a0, a1, p0, p1, wm, bm, wg, xw_o, ylo, yhi):
    agg = a0[...] + a1[...]
    t = jnp.maximum(jnp.dot(agg, wm[...], preferred_element_type=jnp.float32)
                    + bm[...], 0.0)
    xw = jnp.dot(t, wg[...], preferred_element_type=jnp.float32)
    dis, _ = _dis(p0, p1)
    y = dis * xw
    xw_o[...] = xw
    ylo[...] = y[:, :H]
    yhi[...] = y[:, H:]


def _mlp16(a0, a1, p0, p1, wm, bm, wg):
    return _tc_call(
        _mlp16_body,
        [_row_spec(16), _row_spec(16), _row_spec(1), _row_spec(1),
         _full_spec(16, 32), _full_spec(32), _full_spec(32, F)],
        (_row_spec(F), _row_spec(H), _row_spec(H)),
        (_f32(N, F), _f32(NPAD, H), _f32(NPAD, H)),
    )(a0, a1, p0, p1, wm, bm, wg)


def _zcomb_body(a2lo, a2hi, xwh, a4lo, a4hi, xwc, p0, p1, bgf, bgc, hres, z_o):
    dis, dis2 = _dis(p0, p1)
    gh = dis * jnp.concatenate([a2lo[...], a2hi[...]], axis=1) \
        + dis2 * xwh[...] + bgf[...]
    gc = dis * jnp.concatenate([a4lo[...], a4hi[...]], axis=1) \
        + dis2 * xwc[...] + bgc[...]
    z_o[...] = jnp.maximum(gh, 0.0) + jnp.maximum(gc, 0.0) + hres[...]


def _zcomb(a2lo, a2hi, xwh, a4lo, a4hi, xwc, p0, p1, bgf, bgc, hres):
    return _tc_call(
        _zcomb_body,
        [_row_spec(H), _row_spec(H), _row_spec(F), _row_spec(H), _row_spec(H),
         _row_spec(F), _row_spec(1), _row_spec(1), _full_spec(F),
         _full_spec(F), _row_spec(F)],
        _row_spec(F),
        _f32(N, F),
    )(a2lo, a2hi, xwh, a4lo, a4hi, xwc, p0, p1, bgf, bgc, hres)


def _pool1_body(z, bat, w1, b1, w2, b2, gate_o, gmax_o):
    i = pl.program_id(0)
    g1 = jnp.maximum(jnp.dot(z[...], w1[...],
                             preferred_element_type=jnp.float32) + b1[...], 0.0)
    gate = jnp.dot(g1, w2[...], preferred_element_type=jnp.float32) + b2[...]
    gate_o[...] = gate
    bb = bat[...]
    parts = []
    for gi in range(G):
        m = (bb == gi)
        parts.append(jnp.max(jnp.where(m, gate, NEG), axis=0, keepdims=True))
    local = jnp.concatenate(parts, axis=0)

    @pl.when(i == 0)
    def _():
        gmax_o[...] = jnp.full((G, F), NEG, jnp.float32)

    gmax_o[...] = jnp.maximum(gmax_o[...], local)


def _pool1(z, bat_col, w1, b1, w2, b2):
    return _tc_call(
        _pool1_body,
        [_row_spec(F), _row_spec(1), _full_spec(F, 96), _full_spec(96),
         _full_spec(96, F), _full_spec(F)],
        (_row_spec(F), _full_spec(G, F)),
        (_f32(N, F), _f32(G, F)),
    )(z, bat_col, w1, b1, w2, b2)


def _pool2_body(gate, bat, gmax, den_o):
    i = pl.program_id(0)
    gm = gmax[...]
    gm = jnp.where(gm > NEG / 2, gm, 0.0)
    bb = bat[...]
    oh = (bb == lax.broadcasted_iota(jnp.int32, (BLK, G), 1)
          ).astype(jnp.float32)
    gmb = jnp.dot(oh, gm, preferred_element_type=jnp.float32)
    e = jnp.exp(gate[...] - gmb)
    local = lax.dot_general(oh, e, (((0,), (0,)), ((), ())),
                            preferred_element_type=jnp.float32)

    @pl.when(i == 0)
    def _():
        den_o[...] = jnp.zeros((G, F), jnp.float32)

    den_o[...] = den_o[...] + local


def _pool2(gate, bat_col, gmax):
    return _tc_call(
        _pool2_body,
        [_row_spec(F), _row_spec(1), _full_spec(G, F)],
        _full_spec(G, F),
        _f32(G, F),
    )(gate, bat_col, gmax)


def _pool3_body(gate, bat, gmax, den, z, zg_o):
    i = pl.program_id(0)
    gm = gmax[...]
    gm = jnp.where(gm > NEG / 2, gm, 0.0)
    bb = bat[...]
    oh = (bb == lax.broadcasted_iota(jnp.int32, (BLK, G), 1)
          ).astype(jnp.float32)
    gmb = jnp.dot(oh, gm, preferred_element_type=jnp.float32)
    e = jnp.exp(gate[...] - gmb)
    denb = jnp.dot(oh, den[...], preferred_element_type=jnp.float32)
    att = e / (denb + 1e-16)
    local = lax.dot_general(oh, att * z[...], (((0,), (0,)), ((), ())),
                            preferred_element_type=jnp.float32)

    @pl.when(i == 0)
    def _():
        zg_o[...] = jnp.zeros((G, F), jnp.float32)

    zg_o[...] = zg_o[...] + local


def _pool3(gate, bat_col, gmax, den, z):
    return _tc_call(
        _pool3_body,
        [_row_spec(F), _row_spec(1), _full_spec(G, F), _full_spec(G, F),
         _row_spec(F)],
        _full_spec(G, F),
        _f32(G, F),
    )(gate, bat_col, gmax, den, z)


def _pool4_body(bat, zg, wgd, p0, p1, xw_o, ylo, yhi):
    zgd = jnp.dot(zg[...], wgd[...], preferred_element_type=jnp.float32)
    bb = bat[...]
    oh = (bb == lax.broadcasted_iota(jnp.int32, (BLK, G), 1)
          ).astype(jnp.float32)
    xw = jnp.dot(oh, zgd, preferred_element_type=jnp.float32)
    dis, _ = _dis(p0, p1)
    y = dis * xw
    xw_o[...] = xw
    ylo[...] = y[:, :H]
    yhi[...] = y[:, H:]


def _pool4(bat_col, zg, wgd, p0, p1):
    return _tc_call(
        _pool4_body,
        [_row_spec(1), _full_spec(G, F), _full_spec(F, F), _row_spec(1),
         _row_spec(1)],
        (_row_spec(F), _row_spec(H), _row_spec(H)),
        (_f32(N, F), _f32(NPAD, H), _f32(NPAD, H)),
    )(bat_col, zg, wgd, p0, p1)


def _dectail_body(a5lo, a5hi, xw, p0, p1, bgd, pe, res_o, lo, hi):
    dis, dis2 = _dis(p0, p1)
    gd = dis * jnp.concatenate([a5lo[...], a5hi[...]], axis=1) \
        + dis2 * xw[...] + bgd[...]
    zr = jnp.maximum(gd, 0.0)
    res_o[...] = zr
    p = pe[...]
    zin = zr + jnp.concatenate([p, p, p], axis=1)
    lo[...] = zin[:, :H]
    hi[...] = zin[:, H:]


def _dectail(a5lo, a5hi, xw, p0, p1, bgd, pe):
    return _tc_call(
        _dectail_body,
        [_row_spec(H), _row_spec(H), _row_spec(F), _row_spec(1), _row_spec(1),
         _full_spec(F), _row_spec(16)],
        (_row_spec(F), _row_spec(H), _row_spec(H)),
        (_f32(N, F), _f32(NPAD, H), _f32(NPAD, H)),
    )(a5lo, a5hi, xw, p0, p1, bgd, pe)


def _mpnn_body(alo, ahi, w, b, res, lo, hi):
    t = (jnp.dot(alo[...], w[:H, :], preferred_element_type=jnp.float32)
         + jnp.dot(ahi[...], w[H:, :], preferred_element_type=jnp.float32)
         + b[...])
    t = jnp.maximum(t, 0.0)
    if res is not None:
        t = t + res[...]
    lo[...] = t[:, :H]
    hi[...] = t[:, H:]


def _mpnn_step(alo, ahi, w, b, res=None):
    if res is None:
        body = lambda alo, ahi, w, b, lo, hi: _mpnn_body(alo, ahi, w, b, None, lo, hi)
        ins = [_row_spec(H), _row_spec(H), _full_spec(F, F), _full_spec(F)]
        args = (alo, ahi, w, b)
    else:
        body = _mpnn_body
        ins = [_row_spec(H), _row_spec(H), _full_spec(F, F), _full_spec(F),
               _row_spec(F)]
        args = (alo, ahi, w, b, res)
    return _tc_call(
        body, ins,
        (_row_spec(H), _row_spec(H)),
        (_f32(NPAD, H), _f32(NPAD, H)),
    )(*args)


def _final_body(alo, ahi, w, b, o):
    t = (jnp.dot(alo[...], w[:H, :], preferred_element_type=jnp.float32)
         + jnp.dot(ahi[...], w[H:, :], preferred_element_type=jnp.float32)
         + b[...])
    o[...] = jnp.maximum(t, 0.0)


def _final(alo, ahi, w, b):
    return _tc_call(
        _final_body,
        [_row_spec(H), _row_spec(H), _full_spec(F, 128), _full_spec(128)],
        _row_spec(128),
        _f32(N, 128),
    )(alo, ahi, w, b)


def kernel(x1, x2, pe, edge_index, batch, W_ec, b_ec, W_mf, b_mf, W_gf, b_gf,
           W_mc, b_mc, W_gc, b_gc, W_a1, b_a1, W_a2, b_a2, W_gd, b_gd,
           W_d1, b_d1, W_d2, b_d2, W_d3, b_d3):
    src = edge_index[0]
    dst = edge_index[1]
    bat_col = batch.reshape(N, 1)
    z24 = jnp.zeros((ZR, H), jnp.float32)
    z16 = jnp.zeros((ZR, 16), jnp.float32)
    z1v = jnp.zeros((ZR,), jnp.float32)

    p0, p1 = _deg_k()(dst, z1v)
    p0c = p0[:N].reshape(N, 1)
    p1c = p1[:N].reshape(N, 1)

    h_res, hin_lo, hin_hi = _encoder(x1, pe, W_ec, b_ec)
    a1lo, a1hi = _agg48_k()(hin_lo, hin_hi, src, dst, z24)
    xwh, ylo, yhi = _mlp_gcnprep(a1lo[:N], a1hi[:N], p0c, p1c, W_mf, b_mf,
                                 W_gf, F)
    a2lo, a2hi = _agg48_k()(ylo, yhi, src, dst, z24)

    c0, c1 = _agg16_k()(x2, src, dst, z16)
    xwc, yclo, ychi = _mlp16(c0[:N], c1[:N], p0c, p1c, W_mc, b_mc, W_gc)
    a4lo, a4hi = _agg48_k()(yclo, ychi, src, dst, z24)

    z = _zcomb(a2lo[:N], a2hi[:N], xwh, a4lo[:N], a4hi[:N], xwc,
               p0c, p1c, b_gf, b_gc, h_res)

    gate, gmax = _pool1(z, bat_col, W_a1, b_a1, W_a2, b_a2)
    den = _pool2(gate, bat_col, gmax)
    zg = _pool3(gate, bat_col, gmax, den, z)
    xwd, ydlo, ydhi = _pool4(bat_col, zg, W_gd, p0c, p1c)
    a5lo, a5hi = _agg48_k()(ydlo, ydhi, src, dst, z24)
    z1_res, zin_lo, zin_hi = _dectail(a5lo[:N], a5hi[:N], xwd, p0c, p1c,
                                      b_gd, pe)

    a6lo, a6hi = _agg48_k()(zin_lo, zin_hi, src, dst, z24)
    qlo, qhi = _mpnn_step(a6lo[:N], a6hi[:N], W_d1, b_d1)
    a7lo, a7hi = _agg48_k()(qlo, qhi, src, dst, z24)
    rlo, rhi = _mpnn_step(a7lo[:N], a7hi[:N], W_d2, b_d2, res=z1_res)
    a8lo, a8hi = _agg48_k()(rlo, rhi, src, dst, z24)
    return _final(a8lo[:N], a8hi[:N], W_d3, b_d3)


# speedup vs baseline: 13.6296x; 2.7726x over previous
import functools

import jax
import jax.numpy as jnp
from jax import lax
from jax.experimental import pallas as pl
from jax.experimental.pallas import tpu as pltpu, tpu_sc as plsc

N = 50000
E = 1600000
G = 64
F = 48
H = 24
NPAD = 50048
ZR = NPAD // 16
CH = 80
NBUF = 5
BLK = 2000
GRID = N // BLK
NEG = -3.0e38


def _f32(*shape):
    return jax.ShapeDtypeStruct(shape, jnp.float32)


@functools.cache
def _mesh():
    return plsc.VectorSubcoreMesh(core_axis_name="c", subcore_axis_name="s")


def _agg48_body(xlo, xhi, src2, dst2, zeros, out_lo, out_hi, *scr):
    idx_s = scr[0:NBUF]
    idx_d = scr[NBUF:2 * NBUF]
    rows = scr[2 * NBUF:3 * NBUF]
    acc = scr[3 * NBUF]
    sem_i, sem_g, sem_s = scr[3 * NBUF + 1:3 * NBUF + 4]
    c = lax.axis_index("c")
    t = lax.axis_index("s")
    base_r = t * ZR
    pltpu.sync_copy(zeros, acc.at[pl.ds(base_r, ZR)])
    plsc.subcore_barrier()
    nch = (E // 16) // CH
    cbase = t * nch

    def group(g, carry):
        row0 = cbase + g * NBUF
        di = []
        for s in range(NBUF):
            di.append(pltpu.async_copy(src2.at[row0 + s], idx_s[s], sem_i))
            di.append(pltpu.async_copy(dst2.at[row0 + s], idx_d[s], sem_i))
        for d in di:
            d.wait()
        @pl.when(c == 0)
        def _():
            dg = [pltpu.async_copy(xlo.at[idx_s[s]], rows[s], sem_g)
                  for s in range(NBUF)]
            for d in dg:
                d.wait()

        @pl.when(c == 1)
        def _():
            dg = [pltpu.async_copy(xhi.at[idx_s[s]], rows[s], sem_g)
                  for s in range(NBUF)]
            for d in dg:
                d.wait()
        for s in range(NBUF):
            pltpu.sync_copy(rows[s], acc.at[idx_d[s]], add=True)
        return carry

    lax.fori_loop(0, (E // 16) // CH // NBUF, group, 0)
    plsc.subcore_barrier()

    @pl.when(c == 0)
    def _():
        pltpu.sync_copy(acc.at[pl.ds(base_r, ZR)], out_lo.at[pl.ds(base_r, ZR)])

    @pl.when(c == 1)
    def _():
        pltpu.sync_copy(acc.at[pl.ds(base_r, ZR)], out_hi.at[pl.ds(base_r, ZR)])


@functools.cache
def _agg48_k():
  return pl.kernel(
    _agg48_body,
    out_type=(_f32(NPAD, H), _f32(NPAD, H)),
    mesh=_mesh(),
    scratch_types=(
        [pltpu.VMEM((CH,), jnp.int32)] * (2 * NBUF)
        + [pltpu.VMEM((CH, H), jnp.float32)] * NBUF
        + [pltpu.VMEM_SHARED((NPAD, H), jnp.float32)]
        + [pltpu.SemaphoreType.DMA] * 3
    ),
    compiler_params=pltpu.CompilerParams(use_tc_tiling_on_sc=False),
)


def _agg16_body(x2, src2, dst2, zeros, out0, out1, *scr):
    idx_s = scr[0:NBUF]
    idx_d = scr[NBUF:2 * NBUF]
    rows = scr[2 * NBUF:3 * NBUF]
    acc = scr[3 * NBUF]
    sem_i, sem_g, sem_s = scr[3 * NBUF + 1:3 * NBUF + 4]
    c = lax.axis_index("c")
    t = lax.axis_index("s")
    base_r = t * ZR
    pltpu.sync_copy(zeros, acc.at[pl.ds(base_r, ZR)])
    plsc.subcore_barrier()
    nch = (E // 32) // CH
    cbase = c * (nch * 16) + t * nch

    def group(g, carry):
        row0 = cbase + g * NBUF
        di = []
        for s in range(NBUF):
            di.append(pltpu.async_copy(src2.at[row0 + s], idx_s[s], sem_i))
            di.append(pltpu.async_copy(dst2.at[row0 + s], idx_d[s], sem_i))
        for d in di:
            d.wait()
        dg = [pltpu.async_copy(x2.at[idx_s[s]], rows[s], sem_g)
              for s in range(NBUF)]
        for d in dg:
            d.wait()
        for s in range(NBUF):
            pltpu.sync_copy(rows[s], acc.at[idx_d[s]], add=True)
        return carry

    lax.fori_loop(0, (E // 32) // CH // NBUF, group, 0)
    plsc.subcore_barrier()

    @pl.when(c == 0)
    def _():
        pltpu.sync_copy(acc.at[pl.ds(base_r, ZR)], out0.at[pl.ds(base_r, ZR)])

    @pl.when(c == 1)
    def _():
        pltpu.sync_copy(acc.at[pl.ds(base_r, ZR)], out1.at[pl.ds(base_r, ZR)])


@functools.cache
def _agg16_k():
  return pl.kernel(
    _agg16_body,
    out_type=(_f32(NPAD, 16), _f32(NPAD, 16)),
    mesh=_mesh(),
    scratch_types=(
        [pltpu.VMEM((CH,), jnp.int32)] * (2 * NBUF)
        + [pltpu.VMEM((CH, 16), jnp.float32)] * NBUF
        + [pltpu.VMEM_SHARED((NPAD, 16), jnp.float32)]
        + [pltpu.SemaphoreType.DMA] * 3
    ),
    compiler_params=pltpu.CompilerParams(use_tc_tiling_on_sc=False),
)


def _deg_body(dst2, zeros, out0, out1, *scr):
    idx_d = scr[0:NBUF]
    ones = scr[NBUF]
    acc = scr[NBUF + 1]
    sem_i, sem_s = scr[NBUF + 2:NBUF + 4]
    c = lax.axis_index("c")
    t = lax.axis_index("s")
    base_r = t * ZR
    for j in range(CH // 16):
        ones[pl.ds(j * 16, 16)] = jnp.full((16,), 1.0, jnp.float32)
    pltpu.sync_copy(zeros, acc.at[pl.ds(base_r, ZR)])
    plsc.subcore_barrier()
    ept = (E // 32) // CH
    ebase = c * (ept * 16) + t * ept

    def group(g, carry):
        row0 = ebase + g * NBUF
        di = [pltpu.async_copy(dst2.at[row0 + s], idx_d[s], sem_i)
              for s in range(NBUF)]
        for d in di:
            d.wait()
        for s in range(NBUF):
            pltpu.sync_copy(ones, acc.at[idx_d[s]], add=True)
        return carry

    lax.fori_loop(0, ept // NBUF, group, 0)
    plsc.subcore_barrier()

    @pl.when(c == 0)
    def _():
        pltpu.sync_copy(acc.at[pl.ds(base_r, ZR)], out0.at[pl.ds(base_r, ZR)])

    @pl.when(c == 1)
    def _():
        pltpu.sync_copy(acc.at[pl.ds(base_r, ZR)], out1.at[pl.ds(base_r, ZR)])


@functools.cache
def _deg_k():
  return pl.kernel(
    _deg_body,
    out_type=(_f32(NPAD), _f32(NPAD)),
    mesh=_mesh(),
    scratch_types=(
        [pltpu.VMEM((CH,), jnp.int32)] * NBUF
        + [pltpu.VMEM((CH,), jnp.float32)]
        + [pltpu.VMEM_SHARED((NPAD,), jnp.float32)]
        + [pltpu.SemaphoreType.DMA] * 2
    ),
    compiler_params=pltpu.CompilerParams(use_tc_tiling_on_sc=False),
)


def _row_spec(w):
    return pl.BlockSpec((BLK, w), lambda i: (i, 0))


def _full_spec(*shape):
    nd = len(shape)
    return pl.BlockSpec(shape, lambda i, _nd=nd: (0,) * nd)


def _tc_call(body, in_specs, out_specs, out_shape):
    return pl.pallas_call(body, grid=(GRID,), in_specs=in_specs,
                          out_specs=out_specs, out_shape=out_shape)


def _dis(p0_ref, p1_ref):
    deg = p0_ref[...] + p1_ref[...] + 1.0
    return lax.rsqrt(deg), 1.0 / deg


def _enc_body(x1, pe, w, b, h_res, lo, hi):
    h = jnp.dot(x1[...], w[...], preferred_element_type=jnp.float32) + b[...]
    hr = jnp.maximum(h, 0.0)
    h_res[...] = hr
    p = pe[...]
    hin = hr + jnp.concatenate([p, p, p], axis=1)
    lo[...] = hin[:, :H]
    hi[...] = hin[:, H:]


def _encoder(x1, pe, w, b):
    return _tc_call(
        _enc_body,
        [_row_spec(128), _row_spec(16), _full_spec(128, F), _full_spec(F)],
        (_row_spec(F), _row_spec(H), _row_spec(H)),
        (_f32(N, F), _f32(NPAD, H), _f32(NPAD, H)),
    )(x1, pe, w, b)


def _mlp_gcnprep_body(alo, ahi, p0, p1, wm, bm, wg, xw_o, ylo, yhi):
    t = (jnp.dot(alo[...], wm[:H, :], preferred_element_type=jnp.float32)
         + jnp.dot(ahi[...], wm[H:, :], preferred_element_type=jnp.float32)
         + bm[...])
    t = jnp.maximum(t, 0.0)
    xw = jnp.dot(t, wg[...], preferred_element_type=jnp.float32)
    dis, _ = _dis(p0, p1)
    y = dis * xw
    xw_o[...] = xw
    ylo[...] = y[:, :H]
    yhi[...] = y[:, H:]


def _mlp_gcnprep(alo, ahi, p0, p1, wm, bm, wg, kin):
    return _tc_call(
        _mlp_gcnprep_body,
        [_row_spec(H), _row_spec(H), _row_spec(1), _row_spec(1),
         _full_spec(kin, F), _full_spec(F), _full_spec(F, F)],
        (_row_spec(F), _row_spec(H), _row_spec(H)),
        (_f32(N, F), _f32(NPAD, H), _f32(NPAD, H)),
    )(alo, ahi, p0, p1, wm, bm, wg)


def _mlp16_body(a0, a1, p0, p1, wm, bm, wg, xw_o, ylo, yhi):
    agg = a0[...] + a1[...]
    t = jnp.maximum(jnp.dot(agg, wm[...], preferred_element_type=jnp.float32)
                    + bm[...], 0.0)
    xw = jnp.dot(t, wg[...], preferred_element_type=jnp.float32)
    dis, _ = _dis(p0, p1)
    y = dis * xw
    xw_o[...] = xw
    ylo[...] = y[:, :H]
    yhi[...] = y[:, H:]


def _mlp16(a0, a1, p0, p1, wm, bm, wg):
    return _tc_call(
        _mlp16_body,
        [_row_spec(16), _row_spec(16), _row_spec(1), _row_spec(1),
         _full_spec(16, 32), _full_spec(32), _full_spec(32, F)],
        (_row_spec(F), _row_spec(H), _row_spec(H)),
        (_f32(N, F), _f32(NPAD, H), _f32(NPAD, H)),
    )(a0, a1, p0, p1, wm, bm, wg)


def _zcomb_body(a2lo, a2hi, xwh, a4lo, a4hi, xwc, p0, p1, bgf, bgc, hres, z_o):
    dis, dis2 = _dis(p0, p1)
    gh = dis * jnp.concatenate([a2lo[...], a2hi[...]], axis=1) \
        + dis2 * xwh[...] + bgf[...]
    gc = dis * jnp.concatenate([a4lo[...], a4hi[...]], axis=1) \
        + dis2 * xwc[...] + bgc[...]
    z_o[...] = jnp.maximum(gh, 0.0) + jnp.maximum(gc, 0.0) + hres[...]


def _zcomb(a2lo, a2hi, xwh, a4lo, a4hi, xwc, p0, p1, bgf, bgc, hres):
    return _tc_call(
        _zcomb_body,
        [_row_spec(H), _row_spec(H), _row_spec(F), _row_spec(H), _row_spec(H),
         _row_spec(F), _row_spec(1), _row_spec(1), _full_spec(F),
         _full_spec(F), _row_spec(F)],
        _row_spec(F),
        _f32(N, F),
    )(a2lo, a2hi, xwh, a4lo, a4hi, xwc, p0, p1, bgf, bgc, hres)


def _pool1_body(z, bat, w1, b1, w2, b2, gate_o, gmax_o):
    i = pl.program_id(0)
    g1 = jnp.maximum(jnp.dot(z[...], w1[...],
                             preferred_element_type=jnp.float32) + b1[...], 0.0)
    gate = jnp.dot(g1, w2[...], preferred_element_type=jnp.float32) + b2[...]
    gate_o[...] = gate
    bb = bat[...]
    parts = []
    for gi in range(G):
        m = (bb == gi)
        parts.append(jnp.max(jnp.where(m, gate, NEG), axis=0, keepdims=True))
    local = jnp.concatenate(parts, axis=0)

    @pl.when(i == 0)
    def _():
        gmax_o[...] = jnp.full((G, F), NEG, jnp.float32)

    gmax_o[...] = jnp.maximum(gmax_o[...], local)


def _pool1(z, bat_col, w1, b1, w2, b2):
    return _tc_call(
        _pool1_body,
        [_row_spec(F), _row_spec(1), _full_spec(F, 96), _full_spec(96),
         _full_spec(96, F), _full_spec(F)],
        (_row_spec(F), _full_spec(G, F)),
        (_f32(N, F), _f32(G, F)),
    )(z, bat_col, w1, b1, w2, b2)


def _pool2_body(gate, bat, gmax, den_o):
    i = pl.program_id(0)
    gm = gmax[...]
    gm = jnp.where(gm > NEG / 2, gm, 0.0)
    bb = bat[...]
    oh = (bb == lax.broadcasted_iota(jnp.int32, (BLK, G), 1)
          ).astype(jnp.float32)
    gmb = jnp.dot(oh, gm, preferred_element_type=jnp.float32)
    e = jnp.exp(gate[...] - gmb)
    local = lax.dot_general(oh, e, (((0,), (0,)), ((), ())),
                            preferred_element_type=jnp.float32)

    @pl.when(i == 0)
    def _():
        den_o[...] = jnp.zeros((G, F), jnp.float32)

    den_o[...] = den_o[...] + local


def _pool2(gate, bat_col, gmax):
    return _tc_call(
        _pool2_body,
        [_row_spec(F), _row_spec(1), _full_spec(G, F)],
        _full_spec(G, F),
        _f32(G, F),
    )(gate, bat_col, gmax)


def _pool3_body(gate, bat, gmax, den, z, zg_o):
    i = pl.program_id(0)
    gm = gmax[...]
    gm = jnp.where(gm > NEG / 2, gm, 0.0)
    bb = bat[...]
    oh = (bb == lax.broadcasted_iota(jnp.int32, (BLK, G), 1)
          ).astype(jnp.float32)
    gmb = jnp.dot(oh, gm, preferred_element_type=jnp.float32)
    e = jnp.exp(gate[...] - gmb)
    denb = jnp.dot(oh, den[...], preferred_element_type=jnp.float32)
    att = e / (denb + 1e-16)
    local = lax.dot_general(oh, att * z[...], (((0,), (0,)), ((), ())),
                            preferred_element_type=jnp.float32)

    @pl.when(i == 0)
    def _():
        zg_o[...] = jnp.zeros((G, F), jnp.float32)

    zg_o[...] = zg_o[...] + local


def _pool3(gate, bat_col, gmax, den, z):
    return _tc_call(
        _pool3_body,
        [_row_spec(F), _row_spec(1), _full_spec(G, F), _full_spec(G, F),
         _row_spec(F)],
        _full_spec(G, F),
        _f32(G, F),
    )(gate, bat_col, gmax, den, z)


def _pool4_body(bat, zg, wgd, p0, p1, xw_o, ylo, yhi):
    zgd = jnp.dot(zg[...], wgd[...], preferred_element_type=jnp.float32)
    bb = bat[...]
    oh = (bb == lax.broadcasted_iota(jnp.int32, (BLK, G), 1)
          ).astype(jnp.float32)
    xw = jnp.dot(oh, zgd, preferred_element_type=jnp.float32)
    dis, _ = _dis(p0, p1)
    y = dis * xw
    xw_o[...] = xw
    ylo[...] = y[:, :H]
    yhi[...] = y[:, H:]


def _pool4(bat_col, zg, wgd, p0, p1):
    return _tc_call(
        _pool4_body,
        [_row_spec(1), _full_spec(G, F), _full_spec(F, F), _row_spec(1),
         _row_spec(1)],
        (_row_spec(F), _row_spec(H), _row_spec(H)),
        (_f32(N, F), _f32(NPAD, H), _f32(NPAD, H)),
    )(bat_col, zg, wgd, p0, p1)


def _dectail_body(a5lo, a5hi, xw, p0, p1, bgd, pe, res_o, lo, hi):
    dis, dis2 = _dis(p0, p1)
    gd = dis * jnp.concatenate([a5lo[...], a5hi[...]], axis=1) \
        + dis2 * xw[...] + bgd[...]
    zr = jnp.maximum(gd, 0.0)
    res_o[...] = zr
    p = pe[...]
    zin = zr + jnp.concatenate([p, p, p], axis=1)
    lo[...] = zin[:, :H]
    hi[...] = zin[:, H:]


def _dectail(a5lo, a5hi, xw, p0, p1, bgd, pe):
    return _tc_call(
        _dectail_body,
        [_row_spec(H), _row_spec(H), _row_spec(F), _row_spec(1), _row_spec(1),
         _full_spec(F), _row_spec(16)],
        (_row_spec(F), _row_spec(H), _row_spec(H)),
        (_f32(N, F), _f32(NPAD, H), _f32(NPAD, H)),
    )(a5lo, a5hi, xw, p0, p1, bgd, pe)


def _mpnn_body(alo, ahi, w, b, res, lo, hi):
    t = (jnp.dot(alo[...], w[:H, :], preferred_element_type=jnp.float32)
         + jnp.dot(ahi[...], w[H:, :], preferred_element_type=jnp.float32)
         + b[...])
    t = jnp.maximum(t, 0.0)
    if res is not None:
        t = t + res[...]
    lo[...] = t[:, :H]
    hi[...] = t[:, H:]


def _mpnn_step(alo, ahi, w, b, res=None):
    if res is None:
        body = lambda alo, ahi, w, b, lo, hi: _mpnn_body(alo, ahi, w, b, None, lo, hi)
        ins = [_row_spec(H), _row_spec(H), _full_spec(F, F), _full_spec(F)]
        args = (alo, ahi, w, b)
    else:
        body = _mpnn_body
        ins = [_row_spec(H), _row_spec(H), _full_spec(F, F), _full_spec(F),
               _row_spec(F)]
        args = (alo, ahi, w, b, res)
    return _tc_call(
        body, ins,
        (_row_spec(H), _row_spec(H)),
        (_f32(NPAD, H), _f32(NPAD, H)),
    )(*args)


def _final_body(alo, ahi, w, b, o):
    t = (jnp.dot(alo[...], w[:H, :], preferred_element_type=jnp.float32)
         + jnp.dot(ahi[...], w[H:, :], preferred_element_type=jnp.float32)
         + b[...])
    o[...] = jnp.maximum(t, 0.0)


def _final(alo, ahi, w, b):
    return _tc_call(
        _final_body,
        [_row_spec(H), _row_spec(H), _full_spec(F, 128), _full_spec(128)],
        _row_spec(128),
        _f32(N, 128),
    )(alo, ahi, w, b)


def kernel(x1, x2, pe, edge_index, batch, W_ec, b_ec, W_mf, b_mf, W_gf, b_gf,
           W_mc, b_mc, W_gc, b_gc, W_a1, b_a1, W_a2, b_a2, W_gd, b_gd,
           W_d1, b_d1, W_d2, b_d2, W_d3, b_d3):
    src = edge_index[0].reshape(E // CH, CH)
    dst = edge_index[1].reshape(E // CH, CH)
    bat_col = batch.reshape(N, 1)
    z24 = jnp.zeros((ZR, H), jnp.float32)
    z16 = jnp.zeros((ZR, 16), jnp.float32)
    z1v = jnp.zeros((ZR,), jnp.float32)

    p0, p1 = _deg_k()(dst, z1v)
    p0c = p0[:N].reshape(N, 1)
    p1c = p1[:N].reshape(N, 1)

    h_res, hin_lo, hin_hi = _encoder(x1, pe, W_ec, b_ec)
    a1lo, a1hi = _agg48_k()(hin_lo, hin_hi, src, dst, z24)
    xwh, ylo, yhi = _mlp_gcnprep(a1lo[:N], a1hi[:N], p0c, p1c, W_mf, b_mf,
                                 W_gf, F)
    a2lo, a2hi = _agg48_k()(ylo, yhi, src, dst, z24)

    c0, c1 = _agg16_k()(x2, src, dst, z16)
    xwc, yclo, ychi = _mlp16(c0[:N], c1[:N], p0c, p1c, W_mc, b_mc, W_gc)
    a4lo, a4hi = _agg48_k()(yclo, ychi, src, dst, z24)

    z = _zcomb(a2lo[:N], a2hi[:N], xwh, a4lo[:N], a4hi[:N], xwc,
               p0c, p1c, b_gf, b_gc, h_res)

    gate, gmax = _pool1(z, bat_col, W_a1, b_a1, W_a2, b_a2)
    den = _pool2(gate, bat_col, gmax)
    zg = _pool3(gate, bat_col, gmax, den, z)
    xwd, ydlo, ydhi = _pool4(bat_col, zg, W_gd, p0c, p1c)
    a5lo, a5hi = _agg48_k()(ydlo, ydhi, src, dst, z24)
    z1_res, zin_lo, zin_hi = _dectail(a5lo[:N], a5hi[:N], xwd, p0c, p1c,
                                      b_gd, pe)

    a6lo, a6hi = _agg48_k()(zin_lo, zin_hi, src, dst, z24)
    qlo, qhi = _mpnn_step(a6lo[:N], a6hi[:N], W_d1, b_d1)
    a7lo, a7hi = _agg48_k()(qlo, qhi, src, dst, z24)
    rlo, rhi = _mpnn_step(a7lo[:N], a7hi[:N], W_d2, b_d2, res=z1_res)
    a8lo, a8hi = _agg48_k()(rlo, rhi, src, dst, z24)
    return _final(a8lo[:N], a8hi[:N], W_d3, b_d3)


# speedup vs baseline: 19.4551x; 1.4274x over previous
import functools

import jax
import jax.numpy as jnp
from jax import lax
from jax.experimental import pallas as pl
from jax.experimental.pallas import tpu as pltpu, tpu_sc as plsc

N = 50000
E = 1600000
G = 64
F = 48
H = 24
NPAD = 50048
ZR = NPAD // 16
CH = 80
NBUF = 10
NB2 = 5
BLK = 2000
GRID = N // BLK
NEG = -3.0e38


def _f32(*shape):
    return jax.ShapeDtypeStruct(shape, jnp.float32)


@functools.cache
def _mesh():
    return plsc.VectorSubcoreMesh(core_axis_name="c", subcore_axis_name="s")


def _agg48_body(xlo, xhi, src2, dst2, zeros, out_lo, out_hi, *scr):
    idx_s = scr[0:NBUF]
    idx_d = scr[NBUF:2 * NBUF]
    rows = scr[2 * NBUF:3 * NBUF]
    acc = scr[3 * NBUF]
    sem_i, sem_g, sem_s = scr[3 * NBUF + 1:3 * NBUF + 4]
    c = lax.axis_index("c")
    t = lax.axis_index("s")
    base_r = t * ZR
    pltpu.sync_copy(zeros, acc.at[pl.ds(base_r, ZR)])
    plsc.subcore_barrier()
    nch = (E // 16) // CH
    cbase = t * nch

    ngrp = (E // 16) // CH // NBUF

    def group(g, carry):
        @pl.when(g > 0)
        def _():
            for s in range(NBUF):
                pltpu.make_async_copy(zeros.at[pl.ds(0, CH)], rows[s],
                                      sem_s).wait()

        @pl.when(g < ngrp)
        def _():
            row0 = cbase + g * NBUF
            di = []
            for s in range(NBUF):
                di.append(pltpu.async_copy(src2.at[row0 + s], idx_s[s], sem_i))
                di.append(pltpu.async_copy(dst2.at[row0 + s], idx_d[s], sem_i))
            for d in di:
                d.wait()

            @pl.when(c == 0)
            def _():
                for s in range(NBUF):
                    pltpu.async_copy(xlo.at[idx_s[s]], rows[s], sem_g)

            @pl.when(c == 1)
            def _():
                for s in range(NBUF):
                    pltpu.async_copy(xhi.at[idx_s[s]], rows[s], sem_g)
            for s in range(NBUF):
                pltpu.make_async_copy(zeros.at[pl.ds(0, CH)], rows[s],
                                      sem_g).wait()
                pltpu.async_copy(rows[s], acc.at[idx_d[s]], sem_s, add=True)
        return carry

    lax.fori_loop(0, ngrp + 1, group, 0)
    plsc.subcore_barrier()

    @pl.when(c == 0)
    def _():
        pltpu.sync_copy(acc.at[pl.ds(base_r, ZR)], out_lo.at[pl.ds(base_r, ZR)])

    @pl.when(c == 1)
    def _():
        pltpu.sync_copy(acc.at[pl.ds(base_r, ZR)], out_hi.at[pl.ds(base_r, ZR)])


@functools.cache
def _agg48_k():
  return pl.kernel(
    _agg48_body,
    out_type=(_f32(NPAD, H), _f32(NPAD, H)),
    mesh=_mesh(),
    scratch_types=(
        [pltpu.VMEM((CH,), jnp.int32)] * (2 * NBUF)
        + [pltpu.VMEM((CH, H), jnp.float32)] * NBUF
        + [pltpu.VMEM_SHARED((NPAD, H), jnp.float32)]
        + [pltpu.SemaphoreType.DMA] * 3
    ),
    compiler_params=pltpu.CompilerParams(use_tc_tiling_on_sc=False),
)


def _agg16_body(x2, src2, dst2, zeros, out0, out1, *scr):
    idx_s = scr[0:NB2]
    idx_d = scr[NB2:2 * NB2]
    rows = scr[2 * NB2:3 * NB2]
    acc = scr[3 * NB2]
    sem_i, sem_g, sem_s = scr[3 * NB2 + 1:3 * NB2 + 4]
    c = lax.axis_index("c")
    t = lax.axis_index("s")
    base_r = t * ZR
    pltpu.sync_copy(zeros, acc.at[pl.ds(base_r, ZR)])
    plsc.subcore_barrier()
    nch = (E // 32) // CH
    cbase = c * (nch * 16) + t * nch

    def group(g, carry):
        row0 = cbase + g * NB2
        di = []
        for s in range(NB2):
            di.append(pltpu.async_copy(src2.at[row0 + s], idx_s[s], sem_i))
            di.append(pltpu.async_copy(dst2.at[row0 + s], idx_d[s], sem_i))
        for d in di:
            d.wait()
        dg = [pltpu.async_copy(x2.at[idx_s[s]], rows[s], sem_g)
              for s in range(NB2)]
        for d in dg:
            d.wait()
        for s in range(NB2):
            pltpu.sync_copy(rows[s], acc.at[idx_d[s]], add=True)
        return carry

    lax.fori_loop(0, (E // 32) // CH // NB2, group, 0)
    plsc.subcore_barrier()

    @pl.when(c == 0)
    def _():
        pltpu.sync_copy(acc.at[pl.ds(base_r, ZR)], out0.at[pl.ds(base_r, ZR)])

    @pl.when(c == 1)
    def _():
        pltpu.sync_copy(acc.at[pl.ds(base_r, ZR)], out1.at[pl.ds(base_r, ZR)])


@functools.cache
def _agg16_k():
  return pl.kernel(
    _agg16_body,
    out_type=(_f32(NPAD, 16), _f32(NPAD, 16)),
    mesh=_mesh(),
    scratch_types=(
        [pltpu.VMEM((CH,), jnp.int32)] * (2 * NB2)
        + [pltpu.VMEM((CH, 16), jnp.float32)] * NB2
        + [pltpu.VMEM_SHARED((NPAD, 16), jnp.float32)]
        + [pltpu.SemaphoreType.DMA] * 3
    ),
    compiler_params=pltpu.CompilerParams(use_tc_tiling_on_sc=False),
)


def _deg_body(dst2, zeros, out0, out1, *scr):
    idx_d = scr[0:NB2]
    ones = scr[NB2]
    acc = scr[NB2 + 1]
    sem_i, sem_s = scr[NB2 + 2:NB2 + 4]
    c = lax.axis_index("c")
    t = lax.axis_index("s")
    base_r = t * ZR
    for j in range(CH // 16):
        ones[pl.ds(j * 16, 16)] = jnp.full((16,), 1.0, jnp.float32)
    pltpu.sync_copy(zeros, acc.at[pl.ds(base_r, ZR)])
    plsc.subcore_barrier()
    ept = (E // 32) // CH
    ebase = c * (ept * 16) + t * ept

    def group(g, carry):
        row0 = ebase + g * NB2
        di = [pltpu.async_copy(dst2.at[row0 + s], idx_d[s], sem_i)
              for s in range(NB2)]
        for d in di:
            d.wait()
        for s in range(NB2):
            pltpu.sync_copy(ones, acc.at[idx_d[s]], add=True)
        return carry

    lax.fori_loop(0, ept // NB2, group, 0)
    plsc.subcore_barrier()

    @pl.when(c == 0)
    def _():
        pltpu.sync_copy(acc.at[pl.ds(base_r, ZR)], out0.at[pl.ds(base_r, ZR)])

    @pl.when(c == 1)
    def _():
        pltpu.sync_copy(acc.at[pl.ds(base_r, ZR)], out1.at[pl.ds(base_r, ZR)])


@functools.cache
def _deg_k():
  return pl.kernel(
    _deg_body,
    out_type=(_f32(NPAD), _f32(NPAD)),
    mesh=_mesh(),
    scratch_types=(
        [pltpu.VMEM((CH,), jnp.int32)] * NB2
        + [pltpu.VMEM((CH,), jnp.float32)]
        + [pltpu.VMEM_SHARED((NPAD,), jnp.float32)]
        + [pltpu.SemaphoreType.DMA] * 2
    ),
    compiler_params=pltpu.CompilerParams(use_tc_tiling_on_sc=False),
)


def _row_spec(w):
    return pl.BlockSpec((BLK, w), lambda i: (i, 0))


def _full_spec(*shape):
    nd = len(shape)
    return pl.BlockSpec(shape, lambda i, _nd=nd: (0,) * nd)


def _tc_call(body, in_specs, out_specs, out_shape):
    return pl.pallas_call(body, grid=(GRID,), in_specs=in_specs,
                          out_specs=out_specs, out_shape=out_shape)


def _dis(p0_ref, p1_ref):
    deg = p0_ref[...] + p1_ref[...] + 1.0
    return lax.rsqrt(deg), 1.0 / deg


def _enc_body(x1, pe, w, b, h_res, lo, hi):
    h = jnp.dot(x1[...], w[...], preferred_element_type=jnp.float32) + b[...]
    hr = jnp.maximum(h, 0.0)
    h_res[...] = hr
    p = pe[...]
    hin = hr + jnp.concatenate([p, p, p], axis=1)
    lo[...] = hin[:, :H]
    hi[...] = hin[:, H:]


def _encoder(x1, pe, w, b):
    return _tc_call(
        _enc_body,
        [_row_spec(128), _row_spec(16), _full_spec(128, F), _full_spec(F)],
        (_row_spec(F), _row_spec(H), _row_spec(H)),
        (_f32(N, F), _f32(NPAD, H), _f32(NPAD, H)),
    )(x1, pe, w, b)


def _mlp_gcnprep_body(alo, ahi, p0, p1, wm, bm, wg, xw_o, ylo, yhi):
    t = (jnp.dot(alo[...], wm[:H, :], preferred_element_type=jnp.float32)
         + jnp.dot(ahi[...], wm[H:, :], preferred_element_type=jnp.float32)
         + bm[...])
    t = jnp.maximum(t, 0.0)
    xw = jnp.dot(t, wg[...], preferred_element_type=jnp.float32)
    dis, _ = _dis(p0, p1)
    y = dis * xw
    xw_o[...] = xw
    ylo[...] = y[:, :H]
    yhi[...] = y[:, H:]


def _mlp_gcnprep(alo, ahi, p0, p1, wm, bm, wg, kin):
    return _tc_call(
        _mlp_gcnprep_body,
        [_row_spec(H), _row_spec(H), _row_spec(1), _row_spec(1),
         _full_spec(kin, F), _full_spec(F), _full_spec(F, F)],
        (_row_spec(F), _row_spec(H), _row_spec(H)),
        (_f32(N, F), _f32(NPAD, H), _f32(NPAD, H)),
    )(alo, ahi, p0, p1, wm, bm, wg)


def _mlp16_body(a0, a1, p0, p1, wm, bm, wg, xw_o, ylo, yhi):
    agg = a0[...] + a1[...]
    t = jnp.maximum(jnp.dot(agg, wm[...], preferred_element_type=jnp.float32)
                    + bm[...], 0.0)
    xw = jnp.dot(t, wg[...], preferred_element_type=jnp.float32)
    dis, _ = _dis(p0, p1)
    y = dis * xw
    xw_o[...] = xw
    ylo[...] = y[:, :H]
    yhi[...] = y[:, H:]


def _mlp16(a0, a1, p0, p1, wm, bm, wg):
    return _tc_call(
        _mlp16_body,
        [_row_spec(16), _row_spec(16), _row_spec(1), _row_spec(1),
         _full_spec(16, 32), _full_spec(32), _full_spec(32, F)],
        (_row_spec(F), _row_spec(H), _row_spec(H)),
        (_f32(N, F), _f32(NPAD, H), _f32(NPAD, H)),
    )(a0, a1, p0, p1, wm, bm, wg)


def _zcomb_body(a2lo, a2hi, xwh, a4lo, a4hi, xwc, p0, p1, bgf, bgc, hres, z_o):
    dis, dis2 = _dis(p0, p1)
    gh = dis * jnp.concatenate([a2lo[...], a2hi[...]], axis=1) \
        + dis2 * xwh[...] + bgf[...]
    gc = dis * jnp.concatenate([a4lo[...], a4hi[...]], axis=1) \
        + dis2 * xwc[...] + bgc[...]
    z_o[...] = jnp.maximum(gh, 0.0) + jnp.maximum(gc, 0.0) + hres[...]


def _zcomb(a2lo, a2hi, xwh, a4lo, a4hi, xwc, p0, p1, bgf, bgc, hres):
    return _tc_call(
        _zcomb_body,
        [_row_spec(H), _row_spec(H), _row_spec(F), _row_spec(H), _row_spec(H),
         _row_spec(F), _row_spec(1), _row_spec(1), _full_spec(F),
         _full_spec(F), _row_spec(F)],
        _row_spec(F),
        _f32(N, F),
    )(a2lo, a2hi, xwh, a4lo, a4hi, xwc, p0, p1, bgf, bgc, hres)


def _pool1_body(z, bat, w1, b1, w2, b2, gate_o, gmax_o):
    i = pl.program_id(0)
    g1 = jnp.maximum(jnp.dot(z[...], w1[...],
                             preferred_element_type=jnp.float32) + b1[...], 0.0)
    gate = jnp.dot(g1, w2[...], preferred_element_type=jnp.float32) + b2[...]
    gate_o[...] = gate
    bb = bat[...]
    parts = []
    for gi in range(G):
        m = (bb == gi)
        parts.append(jnp.max(jnp.where(m, gate, NEG), axis=0, keepdims=True))
    local = jnp.concatenate(parts, axis=0)

    @pl.when(i == 0)
    def _():
        gmax_o[...] = jnp.full((G, F), NEG, jnp.float32)

    gmax_o[...] = jnp.maximum(gmax_o[...], local)


def _pool1(z, bat_col, w1, b1, w2, b2):
    return _tc_call(
        _pool1_body,
        [_row_spec(F), _row_spec(1), _full_spec(F, 96), _full_spec(96),
         _full_spec(96, F), _full_spec(F)],
        (_row_spec(F), _full_spec(G, F)),
        (_f32(N, F), _f32(G, F)),
    )(z, bat_col, w1, b1, w2, b2)


def _pool2_body(gate, bat, gmax, den_o):
    i = pl.program_id(0)
    gm = gmax[...]
    gm = jnp.where(gm > NEG / 2, gm, 0.0)
    bb = bat[...]
    oh = (bb == lax.broadcasted_iota(jnp.int32, (BLK, G), 1)
          ).astype(jnp.float32)
    gmb = jnp.dot(oh, gm, preferred_element_type=jnp.float32)
    e = jnp.exp(gate[...] - gmb)
    local = lax.dot_general(oh, e, (((0,), (0,)), ((), ())),
                            preferred_element_type=jnp.float32)

    @pl.when(i == 0)
    def _():
        den_o[...] = jnp.zeros((G, F), jnp.float32)

    den_o[...] = den_o[...] + local


def _pool2(gate, bat_col, gmax):
    return _tc_call(
        _pool2_body,
        [_row_spec(F), _row_spec(1), _full_spec(G, F)],
        _full_spec(G, F),
        _f32(G, F),
    )(gate, bat_col, gmax)


def _pool3_body(gate, bat, gmax, den, z, zg_o):
    i = pl.program_id(0)
    gm = gmax[...]
    gm = jnp.where(gm > NEG / 2, gm, 0.0)
    bb = bat[...]
    oh = (bb == lax.broadcasted_iota(jnp.int32, (BLK, G), 1)
          ).astype(jnp.float32)
    gmb = jnp.dot(oh, gm, preferred_element_type=jnp.float32)
    e = jnp.exp(gate[...] - gmb)
    denb = jnp.dot(oh, den[...], preferred_element_type=jnp.float32)
    att = e / (denb + 1e-16)
    local = lax.dot_general(oh, att * z[...], (((0,), (0,)), ((), ())),
                            preferred_element_type=jnp.float32)

    @pl.when(i == 0)
    def _():
        zg_o[...] = jnp.zeros((G, F), jnp.float32)

    zg_o[...] = zg_o[...] + local


def _pool3(gate, bat_col, gmax, den, z):
    return _tc_call(
        _pool3_body,
        [_row_spec(F), _row_spec(1), _full_spec(G, F), _full_spec(G, F),
         _row_spec(F)],
        _full_spec(G, F),
        _f32(G, F),
    )(gate, bat_col, gmax, den, z)


def _pool4_body(bat, zg, wgd, p0, p1, xw_o, ylo, yhi):
    zgd = jnp.dot(zg[...], wgd[...], preferred_element_type=jnp.float32)
    bb = bat[...]
    oh = (bb == lax.broadcasted_iota(jnp.int32, (BLK, G), 1)
          ).astype(jnp.float32)
    xw = jnp.dot(oh, zgd, preferred_element_type=jnp.float32)
    dis, _ = _dis(p0, p1)
    y = dis * xw
    xw_o[...] = xw
    ylo[...] = y[:, :H]
    yhi[...] = y[:, H:]


def _pool4(bat_col, zg, wgd, p0, p1):
    return _tc_call(
        _pool4_body,
        [_row_spec(1), _full_spec(G, F), _full_spec(F, F), _row_spec(1),
         _row_spec(1)],
        (_row_spec(F), _row_spec(H), _row_spec(H)),
        (_f32(N, F), _f32(NPAD, H), _f32(NPAD, H)),
    )(bat_col, zg, wgd, p0, p1)


def _dectail_body(a5lo, a5hi, xw, p0, p1, bgd, pe, res_o, lo, hi):
    dis, dis2 = _dis(p0, p1)
    gd = dis * jnp.concatenate([a5lo[...], a5hi[...]], axis=1) \
        + dis2 * xw[...] + bgd[...]
    zr = jnp.maximum(gd, 0.0)
    res_o[...] = zr
    p = pe[...]
    zin = zr + jnp.concatenate([p, p, p], axis=1)
    lo[...] = zin[:, :H]
    hi[...] = zin[:, H:]


def _dectail(a5lo, a5hi, xw, p0, p1, bgd, pe):
    return _tc_call(
        _dectail_body,
        [_row_spec(H), _row_spec(H), _row_spec(F), _row_spec(1), _row_spec(1),
         _full_spec(F), _row_spec(16)],
        (_row_spec(F), _row_spec(H), _row_spec(H)),
        (_f32(N, F), _f32(NPAD, H), _f32(NPAD, H)),
    )(a5lo, a5hi, xw, p0, p1, bgd, pe)


def _mpnn_body(alo, ahi, w, b, res, lo, hi):
    t = (jnp.dot(alo[...], w[:H, :], preferred_element_type=jnp.float32)
         + jnp.dot(ahi[...], w[H:, :], preferred_element_type=jnp.float32)
         + b[...])
    t = jnp.maximum(t, 0.0)
    if res is not None:
        t = t + res[...]
    lo[...] = t[:, :H]
    hi[...] = t[:, H:]


def _mpnn_step(alo, ahi, w, b, res=None):
    if res is None:
        body = lambda alo, ahi, w, b, lo, hi: _mpnn_body(alo, ahi, w, b, None, lo, hi)
        ins = [_row_spec(H), _row_spec(H), _full_spec(F, F), _full_spec(F)]
        args = (alo, ahi, w, b)
    else:
        body = _mpnn_body
        ins = [_row_spec(H), _row_spec(H), _full_spec(F, F), _full_spec(F),
               _row_spec(F)]
        args = (alo, ahi, w, b, res)
    return _tc_call(
        body, ins,
        (_row_spec(H), _row_spec(H)),
        (_f32(NPAD, H), _f32(NPAD, H)),
    )(*args)


def _final_body(alo, ahi, w, b, o):
    t = (jnp.dot(alo[...], w[:H, :], preferred_element_type=jnp.float32)
         + jnp.dot(ahi[...], w[H:, :], preferred_element_type=jnp.float32)
         + b[...])
    o[...] = jnp.maximum(t, 0.0)


def _final(alo, ahi, w, b):
    return _tc_call(
        _final_body,
        [_row_spec(H), _row_spec(H), _full_spec(F, 128), _full_spec(128)],
        _row_spec(128),
        _f32(N, 128),
    )(alo, ahi, w, b)


def kernel(x1, x2, pe, edge_index, batch, W_ec, b_ec, W_mf, b_mf, W_gf, b_gf,
           W_mc, b_mc, W_gc, b_gc, W_a1, b_a1, W_a2, b_a2, W_gd, b_gd,
           W_d1, b_d1, W_d2, b_d2, W_d3, b_d3):
    src = edge_index[0].reshape(E // CH, CH)
    dst = edge_index[1].reshape(E // CH, CH)
    bat_col = batch.reshape(N, 1)
    z24 = jnp.zeros((ZR, H), jnp.float32)
    z16 = jnp.zeros((ZR, 16), jnp.float32)
    z1v = jnp.zeros((ZR,), jnp.float32)

    p0, p1 = _deg_k()(dst, z1v)
    p0c = p0[:N].reshape(N, 1)
    p1c = p1[:N].reshape(N, 1)

    h_res, hin_lo, hin_hi = _encoder(x1, pe, W_ec, b_ec)
    a1lo, a1hi = _agg48_k()(hin_lo, hin_hi, src, dst, z24)
    xwh, ylo, yhi = _mlp_gcnprep(a1lo[:N], a1hi[:N], p0c, p1c, W_mf, b_mf,
                                 W_gf, F)
    a2lo, a2hi = _agg48_k()(ylo, yhi, src, dst, z24)

    c0, c1 = _agg16_k()(x2, src, dst, z16)
    xwc, yclo, ychi = _mlp16(c0[:N], c1[:N], p0c, p1c, W_mc, b_mc, W_gc)
    a4lo, a4hi = _agg48_k()(yclo, ychi, src, dst, z24)

    z = _zcomb(a2lo[:N], a2hi[:N], xwh, a4lo[:N], a4hi[:N], xwc,
               p0c, p1c, b_gf, b_gc, h_res)

    gate, gmax = _pool1(z, bat_col, W_a1, b_a1, W_a2, b_a2)
    den = _pool2(gate, bat_col, gmax)
    zg = _pool3(gate, bat_col, gmax, den, z)
    xwd, ydlo, ydhi = _pool4(bat_col, zg, W_gd, p0c, p1c)
    a5lo, a5hi = _agg48_k()(ydlo, ydhi, src, dst, z24)
    z1_res, zin_lo, zin_hi = _dectail(a5lo[:N], a5hi[:N], xwd, p0c, p1c,
                                      b_gd, pe)

    a6lo, a6hi = _agg48_k()(zin_lo, zin_hi, src, dst, z24)
    qlo, qhi = _mpnn_step(a6lo[:N], a6hi[:N], W_d1, b_d1)
    a7lo, a7hi = _agg48_k()(qlo, qhi, src, dst, z24)
    rlo, rhi = _mpnn_step(a7lo[:N], a7hi[:N], W_d2, b_d2, res=z1_res)
    a8lo, a8hi = _agg48_k()(rlo, rhi, src, dst, z24)
    return _final(a8lo[:N], a8hi[:N], W_d3, b_d3)
